# baseline (speedup 1.0000x reference)
import functools

import jax
import jax.numpy as jnp
from jax import lax
from jax.experimental import pallas as pl
from jax.experimental.pallas import tpu as pltpu
from jax.experimental.pallas import tpu_sc as plsc

B = 4096
L = 200
EMB = 64
NC = 2
NS = 16
NW = NC * NS
RPW = B // NW
SEQS = 4
ROWS = SEQS * L
NSUP = RPW // SEQS
IR = 4
NBUF = 2


def _make_sc_pool():
    mesh = plsc.VectorSubcoreMesh(core_axis_name="c", subcore_axis_name="s")

    @functools.partial(
        pl.kernel,
        out_type=jax.ShapeDtypeStruct((B, EMB), jnp.float32),
        mesh=mesh,
        compiler_params=pltpu.CompilerParams(use_tc_tiling_on_sc=False,
                                             needs_layout_passes=False),
        scratch_types=[
            pltpu.VMEM((IR, ROWS), jnp.int32),
            pltpu.VMEM((NBUF, ROWS, EMB), jnp.bfloat16),
            pltpu.VMEM((RPW, EMB), jnp.float32),
        ] + [pltpu.SemaphoreType.DMA] * (NBUF + IR),
    )
    def sc_pool(idx_hbm, table_hbm, out_hbm, idx_v, bufs, pooled_v, *sems):
        sems_g = sems[:NBUF]
        sems_i = sems[NBUF:]
        cid = lax.axis_index("c")
        sid = lax.axis_index("s")
        wid = sid * NC + cid
        base = wid * NSUP
        zero = jnp.zeros((16,), jnp.float32)

        for k in range(IR):
            pltpu.async_copy(idx_hbm.at[base + k], idx_v.at[k], sems_i[k])
        for nb in range(NBUF):
            pltpu.make_async_copy(idx_hbm.at[base + nb], idx_v.at[nb],
                                  sems_i[nb]).wait()
            pltpu.async_copy(table_hbm.at[idx_v.at[nb]], bufs.at[nb],
                             sems_g[nb])

        def outer_body(g, carry):
            for su in range(IR):
                s = IR * g + su
                gb = su % NBUF
                pltpu.make_async_copy(table_hbm.at[idx_v.at[su]], bufs.at[gb],
                                      sems_g[gb]).wait()
                for t in range(SEQS):
                    acc = (zero,) * (EMB // 16)

                    def acc_body(i, accs, gb=gb, t=t):
                        a = list(accs)
                        for u in range(8):
                            row = t * L + 8 * i + u
                            for h in range(EMB // 32):
                                v = bufs[gb, row, pl.ds(32 * h, 32)]
                                ev, od = plsc.unpack(
                                    v, format=plsc.PackFormat.INTERLEAVED)
                                a[2 * h] = a[2 * h] + ev
                                a[2 * h + 1] = a[2 * h + 1] + od
                        return tuple(a)

                    acc = lax.fori_loop(0, L // 8, acc_body, acc)
                    r = SEQS * s + t
                    for j in range(EMB // 16):
                        pooled_v[r, pl.ds(16 * j, 16)] = acc[j]

                s_pf = s + IR

                @pl.when(s_pf < NSUP)
                def _(su=su, s_pf=s_pf):
                    pltpu.async_copy(idx_hbm.at[base + s_pf], idx_v.at[su],
                                     sems_i[su])

                s2 = s + NBUF
                ki = (su + NBUF) % IR

                @pl.when(s2 < NSUP)
                def _(gb=gb, ki=ki, s2=s2):
                    pltpu.make_async_copy(idx_hbm.at[base + s2], idx_v.at[ki],
                                          sems_i[ki]).wait()
                    pltpu.async_copy(table_hbm.at[idx_v.at[ki]], bufs.at[gb],
                                     sems_g[gb])
            return carry

        lax.fori_loop(0, NSUP // IR, outer_body, 0)
        pltpu.sync_copy(pooled_v, out_hbm.at[pl.ds(wid * RPW, RPW)])

    return sc_pool


_sc_pool = _make_sc_pool()


def _linear_body(p_ref, w_ref, b_ref, o_ref):
    pooled = p_ref[...] * (1.0 / L)
    acc = jnp.dot(pooled, w_ref[...].T, preferred_element_type=jnp.float32)
    o_ref[...] = jnp.maximum(acc + b_ref[...], 0.0)


def _linear(pooled_sum, w, b):
    return pl.pallas_call(
        _linear_body,
        out_shape=jax.ShapeDtypeStruct((B, EMB), jnp.float32),
    )(pooled_sum, w, b.reshape(1, EMB))


_PERM = tuple(32 * h + 2 * i + p for h in range(EMB // 32)
              for p in range(2) for i in range(16))


def kernel(x, table, W, b):
    idx = x.astype(jnp.int32).reshape(B // SEQS, ROWS)
    pooled_sum = _sc_pool(idx, table.astype(jnp.bfloat16))
    w_perm = W[:, jnp.asarray(_PERM, dtype=jnp.int32)]
    return _linear(pooled_sum, w_perm, b)


# speedup vs baseline: 1.2731x; 1.2731x over previous
import functools

import jax
import jax.numpy as jnp
from jax import lax
from jax.experimental import pallas as pl
from jax.experimental.pallas import tpu as pltpu
from jax.experimental.pallas import tpu_sc as plsc

B = 4096
L = 200
EMB = 64
NC = 2
NS = 16
NW = NC * NS
RPW = B // NW
SEQS = 4
ROWS = SEQS * L
NSUP = RPW // SEQS
IR = 4
NBUF = 2


def _make_sc_pool():
    mesh = plsc.VectorSubcoreMesh(core_axis_name="c", subcore_axis_name="s")

    @functools.partial(
        pl.kernel,
        out_type=jax.ShapeDtypeStruct((B, EMB), jnp.float32),
        mesh=mesh,
        compiler_params=pltpu.CompilerParams(use_tc_tiling_on_sc=False),
        scratch_types=[
            pltpu.VMEM((IR, ROWS), jnp.int32),
            pltpu.VMEM((NBUF, ROWS, EMB), jnp.float32),
            pltpu.VMEM((RPW, EMB), jnp.float32),
        ] + [pltpu.SemaphoreType.DMA] * (NBUF + IR),
    )
    def sc_pool(idx_hbm, table_hbm, out_hbm, idx_v, bufs, pooled_v, *sems):
        sems_g = sems[:NBUF]
        sems_i = sems[NBUF:]
        cid = lax.axis_index("c")
        sid = lax.axis_index("s")
        wid = sid * NC + cid
        base = wid * NSUP
        zero = jnp.zeros((16,), jnp.float32)

        for k in range(IR):
            pltpu.async_copy(idx_hbm.at[base + k], idx_v.at[k], sems_i[k])
        for nb in range(NBUF):
            pltpu.make_async_copy(idx_hbm.at[base + nb], idx_v.at[nb],
                                  sems_i[nb]).wait()
            pltpu.async_copy(table_hbm.at[idx_v.at[nb]], bufs.at[nb],
                             sems_g[nb])

        def outer_body(g, carry):
            for su in range(IR):
                s = IR * g + su
                gb = su % NBUF
                pltpu.make_async_copy(table_hbm.at[idx_v.at[su]], bufs.at[gb],
                                      sems_g[gb]).wait()
                for t in range(SEQS):
                    acc = (zero,) * (EMB // 16)

                    def acc_body(i, accs, gb=gb, t=t):
                        a = list(accs)
                        for u in range(8):
                            row = t * L + 8 * i + u
                            for j in range(EMB // 16):
                                a[j] = a[j] + bufs[gb, row, pl.ds(16 * j, 16)]
                        return tuple(a)

                    acc = lax.fori_loop(0, L // 8, acc_body, acc)
                    r = SEQS * s + t
                    for j in range(EMB // 16):
                        pooled_v[r, pl.ds(16 * j, 16)] = acc[j]

                s_pf = s + IR

                @pl.when(s_pf < NSUP)
                def _(su=su, s_pf=s_pf):
                    pltpu.async_copy(idx_hbm.at[base + s_pf], idx_v.at[su],
                                     sems_i[su])

                s2 = s + NBUF
                ki = (su + NBUF) % IR

                @pl.when(s2 < NSUP)
                def _(gb=gb, ki=ki, s2=s2):
                    pltpu.make_async_copy(idx_hbm.at[base + s2], idx_v.at[ki],
                                          sems_i[ki]).wait()
                    pltpu.async_copy(table_hbm.at[idx_v.at[ki]], bufs.at[gb],
                                     sems_g[gb])
            return carry

        lax.fori_loop(0, NSUP // IR, outer_body, 0)
        pltpu.sync_copy(pooled_v, out_hbm.at[pl.ds(wid * RPW, RPW)])

    return sc_pool


_sc_pool = _make_sc_pool()


def _linear_body(p_ref, w_ref, b_ref, o_ref):
    pooled = p_ref[...] * (1.0 / L)
    acc = jnp.dot(pooled, w_ref[...].T, preferred_element_type=jnp.float32)
    o_ref[...] = jnp.maximum(acc + b_ref[...], 0.0)


def _linear(pooled_sum, w, b):
    return pl.pallas_call(
        _linear_body,
        out_shape=jax.ShapeDtypeStruct((B, EMB), jnp.float32),
    )(pooled_sum, w, b.reshape(1, EMB))


def kernel(x, table, W, b):
    idx = x.astype(jnp.int32).reshape(B // SEQS, ROWS)
    pooled_sum = _sc_pool(idx, table)
    return _linear(pooled_sum, W, b)
